# baseline (speedup 1.0000x reference)
import jax
import jax.numpy as jnp
from jax import lax
from jax.experimental import pallas as pl
from jax.experimental.pallas import tpu as pltpu
from jax.experimental.pallas import tpu_sc as plsc

_N = 10000
_NPAD = 10008
_E = 320000
_NC = 2
_NS = 16
_NW = _NC * _NS
_EPT = _E // _NW
_CHUNK = 128
_NCHUNK = 80
_EPTP = _NCHUNK * _CHUNK
_RPT = _N // _NS


def _sc_mesh():
    return plsc.VectorSubcoreMesh(
        core_axis_name="c", subcore_axis_name="s",
        num_cores=_NC, num_subcores=_NS)


def _deg_body(dst_hbm, out_hbm, dst_v, acc_v):
    c = lax.axis_index("c")
    s = lax.axis_index("s")
    wid = c * _NS + s
    pltpu.sync_copy(dst_hbm.at[pl.ds(wid * _EPT, _EPT)], dst_v)
    zeros = jnp.zeros((16,), jnp.float32)

    def zero_step(i, carry):
        acc_v[pl.ds(i * 16, 16)] = zeros
        return carry
    lax.fori_loop(0, _N // 16, zero_step, 0)

    ones = jnp.ones((16,), jnp.float32)

    def step(i, carry):
        idx = dst_v[pl.ds(i * 16, 16)]
        plsc.addupdate_scatter(acc_v, [idx], ones)
        return carry
    lax.fori_loop(0, _EPT // 16, step, 0)
    pltpu.sync_copy(acc_v, out_hbm.at[wid])


def _deg_kernel(dst):
    return pl.kernel(
        _deg_body,
        out_type=jax.ShapeDtypeStruct((_NW, _N), jnp.float32),
        mesh=_sc_mesh(),
        scratch_types=[
            pltpu.VMEM((_EPT,), jnp.int32),
            pltpu.VMEM((_N,), jnp.float32),
        ],
        compiler_params=pltpu.CompilerParams(needs_layout_passes=False, use_tc_tiling_on_sc=False),
    )(dst)


def _make_agg(H):
    def body(hs_hbm, src_hbm, dst_hbm, out_hbm, src_v, dst_v, rows0, rows1,
             rows2, rows3, acc_sh, gsem0, gsem1, gsem2, gsem3,
             ssem0, ssem1, ssem2, ssem3):
        c = lax.axis_index("c")
        s = lax.axis_index("s")
        wid = c * _NS + s
        pltpu.sync_copy(src_hbm.at[wid], src_v)
        pltpu.sync_copy(dst_hbm.at[wid], dst_v)
        pltpu.sync_copy(hs_hbm.at[pl.ds(s * _RPT, _RPT)],
                        acc_sh.at[pl.ds(s * _RPT, _RPT)])
        plsc.subcore_barrier()

        bufs = (rows0, rows1, rows2, rows3)
        gsems = (gsem0, gsem1, gsem2, gsem3)
        ssems = (ssem0, ssem1, ssem2, ssem3)

        def gather(k, j):
            pltpu.async_copy(hs_hbm.at[src_v.at[k]], bufs[j], gsems[j])

        def wait_gather(k, j):
            pltpu.make_async_copy(
                hs_hbm.at[src_v.at[k]], bufs[j], gsems[j]).wait()

        def scatter(k, j):
            pltpu.async_copy(bufs[j], acc_sh.at[dst_v.at[k]], ssems[j],
                             add=True)

        def wait_scatter(k, j):
            pltpu.make_async_copy(
                bufs[j], acc_sh.at[dst_v.at[k]], ssems[j]).wait()

        for j in range(4):
            gather(j, j)

        def step(i, carry):
            k = 4 * i
            for j in range(4):
                wait_gather(k + j, j)
                scatter(k + j, j)
            for j in range(4):
                wait_scatter(k + j, j)
                gather(k + 4 + j, j)
            return carry
        lax.fori_loop(0, _NCHUNK // 4 - 1, step, 0)
        klast = _NCHUNK - 4
        for j in range(4):
            wait_gather(klast + j, j)
            scatter(klast + j, j)
        for j in range(4):
            wait_scatter(klast + j, j)
        plsc.subcore_barrier()
        pltpu.sync_copy(acc_sh.at[pl.ds(s * _RPT, _RPT)],
                        out_hbm.at[c, pl.ds(s * _RPT, _RPT)])

    return pl.kernel(
        body,
        out_type=jax.ShapeDtypeStruct((_NC, _N, H), jnp.float32),
        mesh=_sc_mesh(),
        scratch_types=[
            pltpu.VMEM((_NCHUNK, _CHUNK), jnp.int32),
            pltpu.VMEM((_NCHUNK, _CHUNK), jnp.int32),
            pltpu.VMEM((_CHUNK, H), jnp.float32),
            pltpu.VMEM((_CHUNK, H), jnp.float32),
            pltpu.VMEM((_CHUNK, H), jnp.float32),
            pltpu.VMEM((_CHUNK, H), jnp.float32),
            pltpu.VMEM_SHARED((_NPAD, H), jnp.float32),
        ] + [pltpu.SemaphoreType.DMA] * 8,
        compiler_params=pltpu.CompilerParams(needs_layout_passes=False, use_tc_tiling_on_sc=False),
    )


_agg_cache = {}


def _agg(H, hs, src, dst):
    if H not in _agg_cache:
        _agg_cache[H] = _make_agg(H)
    return _agg_cache[H](hs, src, dst)


def _tc1_body(pt_ref, x_ref, w_ref, hs_ref, dinv_ref):
    deg = jnp.sum(pt_ref[...], axis=1, keepdims=True) + 1.0
    dinv = lax.rsqrt(deg)
    h = jnp.dot(x_ref[...], w_ref[...], preferred_element_type=jnp.float32)
    hs_ref[...] = h * dinv
    dinv_ref[...] = dinv


def _tc1(parts_t, x, w):
    return pl.pallas_call(
        _tc1_body,
        out_shape=[
            jax.ShapeDtypeStruct((_N, w.shape[1]), jnp.float32),
            jax.ShapeDtypeStruct((_N, 1), jnp.float32),
        ],
    )(parts_t, x, w)


def _tc_mid_body(seg_ref, hs_ref, dinv_ref, b_ref, w_ref, out_ref):
    dinv = dinv_ref[...]
    ssum = seg_ref[0] + seg_ref[1] - hs_ref[...]
    act = jnp.maximum(dinv * ssum + b_ref[...], 0.0)
    h = jnp.dot(act, w_ref[...], preferred_element_type=jnp.float32)
    out_ref[...] = h * dinv


def _tc_mid(seg, hs, dinv, b, w):
    return pl.pallas_call(
        _tc_mid_body,
        out_shape=jax.ShapeDtypeStruct((_N, w.shape[1]), jnp.float32),
    )(seg, hs, dinv, b, w)


def _tc_fin_body(seg_ref, hs_ref, dinv_ref, b_ref, out_ref):
    o = dinv_ref[...] * (seg_ref[0] + seg_ref[1] - hs_ref[...]) + b_ref[...]
    m = jnp.max(o, axis=1, keepdims=True)
    lse = jnp.log(jnp.sum(jnp.exp(o - m), axis=1, keepdims=True)) + m
    out_ref[...] = o - lse


def _tc_fin(seg, hs, dinv, b):
    return pl.pallas_call(
        _tc_fin_body,
        out_shape=jax.ShapeDtypeStruct((_N, b.shape[0]), jnp.float32),
    )(seg, hs, dinv, b)


def _pad_idx(v):
    pad = jnp.full((_NW, _EPTP - _EPT), _N, jnp.int32)
    return jnp.concatenate([v.reshape(_NW, _EPT), pad], axis=1).reshape(
        _NW, _NCHUNK, _CHUNK)


def kernel(x, edge_index, W1, b1, W2, b2, W3, b3):
    src = edge_index[0].astype(jnp.int32)
    dst = edge_index[1].astype(jnp.int32)
    src3 = _pad_idx(src)
    dst3 = _pad_idx(dst)
    deg_parts = _deg_kernel(dst)
    parts_t = deg_parts.T
    hs1, dinv = _tc1(parts_t, x, W1)
    seg1 = _agg(64, jnp.pad(hs1, ((0, _NPAD - _N), (0, 0))), src3, dst3)
    hs2 = _tc_mid(seg1, hs1, dinv, b1, W2)
    seg2 = _agg(64, jnp.pad(hs2, ((0, _NPAD - _N), (0, 0))), src3, dst3)
    hs3 = _tc_mid(seg2, hs2, dinv, b2, W3)
    seg3 = _agg(16, jnp.pad(hs3, ((0, _NPAD - _N), (0, 0))), src3, dst3)
    return _tc_fin(seg3, hs3, dinv, b3)


# speedup vs baseline: 1.3134x; 1.3134x over previous
import jax
import jax.numpy as jnp
from jax import lax
from jax.experimental import pallas as pl
from jax.experimental.pallas import tpu as pltpu
from jax.experimental.pallas import tpu_sc as plsc

_N = 10000
_NPAD = 10016
_E = 320000
_NC = 2
_NS = 16
_NW = _NC * _NS
_EPT = _E // _NW
_CHUNK = 128
_NCHUNK = 79
_EPTP = _NCHUNK * _CHUNK
_RPT = _NPAD // _NS


def _sc_mesh():
    return plsc.VectorSubcoreMesh(
        core_axis_name="c", subcore_axis_name="s",
        num_cores=_NC, num_subcores=_NS)


_sc_params = pltpu.CompilerParams(
    needs_layout_passes=False, use_tc_tiling_on_sc=False)


def _deg_body(dst_hbm, out_hbm, dst_v, acc_v):
    c = lax.axis_index("c")
    s = lax.axis_index("s")
    wid = c * _NS + s
    pltpu.sync_copy(dst_hbm.at[pl.ds(wid * _EPT, _EPT)], dst_v)
    zeros = jnp.zeros((16,), jnp.float32)

    def zero_step(i, carry):
        acc_v[pl.ds(i * 16, 16)] = zeros
        return carry
    lax.fori_loop(0, _NPAD // 16, zero_step, 0)

    ones = jnp.ones((16,), jnp.float32)

    def step(i, carry):
        idx = dst_v[pl.ds(i * 16, 16)]
        plsc.addupdate_scatter(acc_v, [idx], ones)
        return carry
    lax.fori_loop(0, _EPT // 16, step, 0)
    pltpu.sync_copy(acc_v, out_hbm.at[wid])


def _deg_kernel(dst):
    return pl.kernel(
        _deg_body,
        out_type=jax.ShapeDtypeStruct((_NW, _NPAD), jnp.float32),
        mesh=_sc_mesh(),
        scratch_types=[
            pltpu.VMEM((_EPT,), jnp.int32),
            pltpu.VMEM((_NPAD,), jnp.float32),
        ],
        compiler_params=_sc_params,
    )(dst)


def _make_agg(H):
    def body(hs_hbm, src_hbm, dst_hbm, out_hbm, src_v, dst_v, rows0, rows1,
             acc_sh, sem0, sem1):
        c = lax.axis_index("c")
        s = lax.axis_index("s")
        wid = c * _NS + s
        pltpu.sync_copy(src_hbm.at[wid], src_v)
        pltpu.sync_copy(dst_hbm.at[wid], dst_v)
        pltpu.sync_copy(hs_hbm.at[pl.ds(s * _RPT, _RPT)],
                        acc_sh.at[pl.ds(s * _RPT, _RPT)])
        plsc.subcore_barrier()

        pltpu.async_copy(hs_hbm.at[src_v.at[0]], rows0, sem0)

        def step(i, carry):
            k0 = 2 * i
            k1 = 2 * i + 1
            k2 = 2 * i + 2
            pltpu.async_copy(hs_hbm.at[src_v.at[k1]], rows1, sem1)
            pltpu.make_async_copy(hs_hbm.at[src_v.at[k0]], rows0, sem0).wait()
            pltpu.sync_copy(rows0, acc_sh.at[dst_v.at[k0]], add=True)
            pltpu.async_copy(hs_hbm.at[src_v.at[k2]], rows0, sem0)
            pltpu.make_async_copy(hs_hbm.at[src_v.at[k1]], rows1, sem1).wait()
            pltpu.sync_copy(rows1, acc_sh.at[dst_v.at[k1]], add=True)
            return carry
        lax.fori_loop(0, (_NCHUNK - 1) // 2, step, 0)
        last = _NCHUNK - 1
        pltpu.make_async_copy(hs_hbm.at[src_v.at[last]], rows0, sem0).wait()
        pltpu.sync_copy(rows0, acc_sh.at[dst_v.at[last]], add=True)
        plsc.subcore_barrier()
        pltpu.sync_copy(acc_sh.at[pl.ds(s * _RPT, _RPT)],
                        out_hbm.at[c, pl.ds(s * _RPT, _RPT)])

    return pl.kernel(
        body,
        out_type=jax.ShapeDtypeStruct((_NC, _NPAD, H), jnp.float32),
        mesh=_sc_mesh(),
        scratch_types=[
            pltpu.VMEM((_NCHUNK, _CHUNK), jnp.int32),
            pltpu.VMEM((_NCHUNK, _CHUNK), jnp.int32),
            pltpu.VMEM((_CHUNK, H), jnp.float32),
            pltpu.VMEM((_CHUNK, H), jnp.float32),
            pltpu.VMEM_SHARED((_NPAD, H), jnp.float32),
            pltpu.SemaphoreType.DMA,
            pltpu.SemaphoreType.DMA,
        ],
        compiler_params=_sc_params,
    )


_agg_cache = {}


def _agg(H, hs, src, dst):
    if H not in _agg_cache:
        _agg_cache[H] = _make_agg(H)
    return _agg_cache[H](hs, src, dst)


def _tc1_body(pt_ref, x_ref, w_ref, hs_ref, dinv_ref):
    deg = jnp.sum(pt_ref[...], axis=1, keepdims=True) + 1.0
    dinv = lax.rsqrt(deg)
    h = jnp.dot(x_ref[...], w_ref[...], preferred_element_type=jnp.float32)
    hs_ref[...] = h * dinv
    dinv_ref[...] = dinv


def _tc1(parts_t, x, w):
    return pl.pallas_call(
        _tc1_body,
        out_shape=[
            jax.ShapeDtypeStruct((_NPAD, w.shape[1]), jnp.float32),
            jax.ShapeDtypeStruct((_NPAD, 1), jnp.float32),
        ],
    )(parts_t, x, w)


def _tc_mid_body(seg_ref, hs_ref, dinv_ref, b_ref, w_ref, out_ref):
    dinv = dinv_ref[...]
    ssum = seg_ref[0] + seg_ref[1] - hs_ref[...]
    act = jnp.maximum(dinv * ssum + b_ref[...], 0.0)
    h = jnp.dot(act, w_ref[...], preferred_element_type=jnp.float32)
    out_ref[...] = h * dinv


def _tc_mid(seg, hs, dinv, b, w):
    return pl.pallas_call(
        _tc_mid_body,
        out_shape=jax.ShapeDtypeStruct((_NPAD, w.shape[1]), jnp.float32),
    )(seg, hs, dinv, b, w)


def _tc_fin_body(seg_ref, hs_ref, dinv_ref, b_ref, out_ref):
    o = dinv_ref[...] * (seg_ref[0] + seg_ref[1] - hs_ref[...]) + b_ref[...]
    m = jnp.max(o, axis=1, keepdims=True)
    lse = jnp.log(jnp.sum(jnp.exp(o - m), axis=1, keepdims=True)) + m
    out_ref[...] = o - lse


def _tc_fin(seg, hs, dinv, b):
    return pl.pallas_call(
        _tc_fin_body,
        out_shape=jax.ShapeDtypeStruct((_NPAD, b.shape[0]), jnp.float32),
    )(seg, hs, dinv, b)[:_N]


def _pad_idx(v):
    pad = jnp.full((_NW, _EPTP - _EPT), _N, jnp.int32)
    return jnp.concatenate([v.reshape(_NW, _EPT), pad], axis=1).reshape(
        _NW, _NCHUNK, _CHUNK)


def kernel(x, edge_index, W1, b1, W2, b2, W3, b3):
    src = edge_index[0].astype(jnp.int32)
    dst = edge_index[1].astype(jnp.int32)
    src3 = _pad_idx(src)
    dst3 = _pad_idx(dst)
    xp = jnp.pad(x, ((0, _NPAD - _N), (0, 0)))
    deg_parts = _deg_kernel(dst)
    parts_t = deg_parts.T
    hs1, dinv = _tc1(parts_t, xp, W1)
    seg1 = _agg(64, hs1, src3, dst3)
    hs2 = _tc_mid(seg1, hs1, dinv, b1, W2)
    seg2 = _agg(64, hs2, src3, dst3)
    hs3 = _tc_mid(seg2, hs2, dinv, b2, W3)
    seg3 = _agg(16, hs3, src3, dst3)
    return _tc_fin(seg3, hs3, dinv, b3)


# speedup vs baseline: 1.8907x; 1.4395x over previous
import jax
import jax.numpy as jnp
from jax import lax
from jax.experimental import pallas as pl
from jax.experimental.pallas import tpu as pltpu
from jax.experimental.pallas import tpu_sc as plsc

_N = 10000
_NPAD = 10016
_E = 320000
_NC = 2
_NS = 16
_NW = _NC * _NS
_EPT = _E // _NW
_CHUNK = 128
_NCHUNK = 79
_EPTP = _NCHUNK * _CHUNK
_RPT = _NPAD // _NS


def _sc_mesh():
    return plsc.VectorSubcoreMesh(
        core_axis_name="c", subcore_axis_name="s",
        num_cores=_NC, num_subcores=_NS)


_sc_params = pltpu.CompilerParams(
    needs_layout_passes=False, use_tc_tiling_on_sc=False)


def _deg_body(dst_hbm, out_hbm, dst_v, acc_v):
    c = lax.axis_index("c")
    s = lax.axis_index("s")
    wid = c * _NS + s
    pltpu.sync_copy(dst_hbm.at[pl.ds(wid * _EPT, _EPT)], dst_v)
    zeros = jnp.zeros((16,), jnp.float32)

    def zero_step(i, carry):
        acc_v[pl.ds(i * 16, 16)] = zeros
        return carry
    lax.fori_loop(0, _NPAD // 16, zero_step, 0)

    ones = jnp.ones((16,), jnp.float32)

    def step(i, carry):
        idx = dst_v[pl.ds(i * 16, 16)]
        plsc.addupdate_scatter(acc_v, [idx], ones)
        return carry
    lax.fori_loop(0, _EPT // 16, step, 0)
    pltpu.sync_copy(acc_v, out_hbm.at[wid])


def _deg_kernel(dst):
    return pl.kernel(
        _deg_body,
        out_type=jax.ShapeDtypeStruct((_NW, _NPAD), jnp.float32),
        mesh=_sc_mesh(),
        scratch_types=[
            pltpu.VMEM((_EPT,), jnp.int32),
            pltpu.VMEM((_NPAD,), jnp.float32),
        ],
        compiler_params=_sc_params,
    )(dst)


def _make_agg(H):
    def body(hs_hbm, src_hbm, dst_hbm, out_hbm, src_v, dst_v, rows0, rows1,
             tab_sh, acc_sh, sem0, sem1):
        c = lax.axis_index("c")
        s = lax.axis_index("s")
        wid = c * _NS + s
        pltpu.sync_copy(src_hbm.at[wid], src_v)
        pltpu.sync_copy(dst_hbm.at[wid], dst_v)
        pltpu.sync_copy(hs_hbm.at[pl.ds(s * _RPT, _RPT)],
                        tab_sh.at[pl.ds(s * _RPT, _RPT)])
        pltpu.sync_copy(hs_hbm.at[pl.ds(s * _RPT, _RPT)],
                        acc_sh.at[pl.ds(s * _RPT, _RPT)])
        plsc.subcore_barrier()

        pltpu.async_copy(tab_sh.at[src_v.at[0]], rows0, sem0)

        def step(i, carry):
            k0 = 2 * i
            k1 = 2 * i + 1
            k2 = 2 * i + 2
            pltpu.async_copy(tab_sh.at[src_v.at[k1]], rows1, sem1)
            pltpu.make_async_copy(tab_sh.at[src_v.at[k0]], rows0, sem0).wait()
            pltpu.sync_copy(rows0, acc_sh.at[dst_v.at[k0]], add=True)
            pltpu.async_copy(tab_sh.at[src_v.at[k2]], rows0, sem0)
            pltpu.make_async_copy(tab_sh.at[src_v.at[k1]], rows1, sem1).wait()
            pltpu.sync_copy(rows1, acc_sh.at[dst_v.at[k1]], add=True)
            return carry
        lax.fori_loop(0, (_NCHUNK - 1) // 2, step, 0)
        last = _NCHUNK - 1
        pltpu.make_async_copy(tab_sh.at[src_v.at[last]], rows0, sem0).wait()
        pltpu.sync_copy(rows0, acc_sh.at[dst_v.at[last]], add=True)
        plsc.subcore_barrier()
        pltpu.sync_copy(acc_sh.at[pl.ds(s * _RPT, _RPT)],
                        out_hbm.at[c, pl.ds(s * _RPT, _RPT)])

    return pl.kernel(
        body,
        out_type=jax.ShapeDtypeStruct((_NC, _NPAD, H), jnp.float32),
        mesh=_sc_mesh(),
        scratch_types=[
            pltpu.VMEM((_NCHUNK, _CHUNK), jnp.int32),
            pltpu.VMEM((_NCHUNK, _CHUNK), jnp.int32),
            pltpu.VMEM((_CHUNK, H), jnp.float32),
            pltpu.VMEM((_CHUNK, H), jnp.float32),
            pltpu.VMEM_SHARED((_NPAD, H), jnp.float32),
            pltpu.VMEM_SHARED((_NPAD, H), jnp.float32),
            pltpu.SemaphoreType.DMA,
            pltpu.SemaphoreType.DMA,
        ],
        compiler_params=_sc_params,
    )


_agg_cache = {}


def _agg(H, hs, src, dst):
    if H not in _agg_cache:
        _agg_cache[H] = _make_agg(H)
    return _agg_cache[H](hs, src, dst)


def _tc1_body(pt_ref, x_ref, w_ref, hs_ref, dinv_ref):
    deg = jnp.sum(pt_ref[...], axis=1, keepdims=True) + 1.0
    dinv = lax.rsqrt(deg)
    h = jnp.dot(x_ref[...], w_ref[...], preferred_element_type=jnp.float32)
    hs_ref[...] = h * dinv
    dinv_ref[...] = dinv


def _tc1(parts_t, x, w):
    return pl.pallas_call(
        _tc1_body,
        out_shape=[
            jax.ShapeDtypeStruct((_NPAD, w.shape[1]), jnp.float32),
            jax.ShapeDtypeStruct((_NPAD, 1), jnp.float32),
        ],
    )(parts_t, x, w)


def _tc_mid_body(seg_ref, hs_ref, dinv_ref, b_ref, w_ref, out_ref):
    dinv = dinv_ref[...]
    ssum = seg_ref[0] + seg_ref[1] - hs_ref[...]
    act = jnp.maximum(dinv * ssum + b_ref[...], 0.0)
    h = jnp.dot(act, w_ref[...], preferred_element_type=jnp.float32)
    out_ref[...] = h * dinv


def _tc_mid(seg, hs, dinv, b, w):
    return pl.pallas_call(
        _tc_mid_body,
        out_shape=jax.ShapeDtypeStruct((_NPAD, w.shape[1]), jnp.float32),
    )(seg, hs, dinv, b, w)


def _tc_fin_body(seg_ref, hs_ref, dinv_ref, b_ref, out_ref):
    o = dinv_ref[...] * (seg_ref[0] + seg_ref[1] - hs_ref[...]) + b_ref[...]
    m = jnp.max(o, axis=1, keepdims=True)
    lse = jnp.log(jnp.sum(jnp.exp(o - m), axis=1, keepdims=True)) + m
    out_ref[...] = o - lse


def _tc_fin(seg, hs, dinv, b):
    return pl.pallas_call(
        _tc_fin_body,
        out_shape=jax.ShapeDtypeStruct((_NPAD, b.shape[0]), jnp.float32),
    )(seg, hs, dinv, b)[:_N]


def _pad_idx(v):
    pad = jnp.full((_NW, _EPTP - _EPT), _N, jnp.int32)
    return jnp.concatenate([v.reshape(_NW, _EPT), pad], axis=1).reshape(
        _NW, _NCHUNK, _CHUNK)


def kernel(x, edge_index, W1, b1, W2, b2, W3, b3):
    src = edge_index[0].astype(jnp.int32)
    dst = edge_index[1].astype(jnp.int32)
    src3 = _pad_idx(src)
    dst3 = _pad_idx(dst)
    xp = jnp.pad(x, ((0, _NPAD - _N), (0, 0)))
    deg_parts = _deg_kernel(dst)
    parts_t = deg_parts.T
    hs1, dinv = _tc1(parts_t, xp, W1)
    seg1 = _agg(64, hs1, src3, dst3)
    hs2 = _tc_mid(seg1, hs1, dinv, b1, W2)
    seg2 = _agg(64, hs2, src3, dst3)
    hs3 = _tc_mid(seg2, hs2, dinv, b2, W3)
    seg3 = _agg(16, hs3, src3, dst3)
    return _tc_fin(seg3, hs3, dinv, b3)
